# baseline (speedup 1.0000x reference)
import functools

import jax
import jax.numpy as jnp
from jax import lax
from jax.experimental import pallas as pl
from jax.experimental.pallas import tpu as pltpu
from jax.experimental.pallas import tpu_sc as plsc

NUM_CORES = 2
NUM_SUBCORES = 16
NUM_WORKERS = NUM_CORES * NUM_SUBCORES
LANES = 16
GCHUNK = 40
RP_ROWS = 4000


def _repack_tc(table):
    V, E = table.shape

    def rp(i_ref, o_ref):
        even = i_ref[pl.Slice(0, RP_ROWS // 2, 2), :]
        odd = i_ref[pl.Slice(1, RP_ROWS // 2, 2), :]
        o_ref[...] = jnp.concatenate([even, odd], axis=1)

    return pl.pallas_call(
        rp,
        grid=(V // RP_ROWS,),
        in_specs=[pl.BlockSpec((RP_ROWS, E), lambda i: (i, 0))],
        out_specs=pl.BlockSpec((RP_ROWS // 2, 2 * E), lambda i: (i, 0)),
        out_shape=jax.ShapeDtypeStruct((V // 2, 2 * E), jnp.float32),
    )(table)


def _build_sc_kernel(B, S, E, V):
    BPW = B // NUM_WORKERS
    RPW = BPW * S
    PAIR = 2 * E
    NG = S // GCHUNK
    GOFFS = [g * LANES for g in range(S // LANES)]
    if S % LANES:
        GOFFS.append(S - LANES)

    mesh = plsc.VectorSubcoreMesh(
        core_axis_name="c", subcore_axis_name="s",
        num_cores=NUM_CORES, num_subcores=NUM_SUBCORES)

    @functools.partial(
        pl.kernel,
        out_type=jax.ShapeDtypeStruct((B, S, E), jnp.float32),
        mesh=mesh,
        scratch_types=[
            pltpu.VMEM((RPW,), jnp.int32),
            pltpu.VMEM((RPW,), jnp.int32),
            pltpu.VMEM((2, S, PAIR), jnp.float32),
            pltpu.VMEM((1, S, E), jnp.float32),
            pltpu.VMEM((S, E), jnp.float32),
            pltpu.SemaphoreType.DMA,
            pltpu.SemaphoreType.DMA,
            pltpu.SemaphoreType.DMA,
        ],
    )
    def k(x1, table2, pos_hbm, out3, idx_all, pidx_all, grows, res, pos_v,
          gsem0, gsem1, osem):
        c = lax.axis_index("c")
        s = lax.axis_index("s")
        wid = s * NUM_CORES + c
        base = pl.multiple_of(wid * RPW, RPW)
        base_b = pl.multiple_of(wid * BPW, BPW)
        gsems = (gsem0, gsem1)
        pltpu.sync_copy(pos_hbm, pos_v)
        pltpu.sync_copy(x1.at[pl.ds(base, RPW)], idx_all)
        for g in range(RPW // LANES):
            sl = pl.ds(g * LANES, LANES)
            pidx_all[sl] = jax.lax.shift_right_logical(idx_all[sl], 1)

        def gather_start(t, b):
            for p in range(NG):
                pltpu.async_copy(
                    table2.at[pidx_all.at[pl.ds(t * S + p * GCHUNK, GCHUNK)]],
                    grows.at[b, pl.ds(p * GCHUNK, GCHUNK)],
                    gsems[b])

        def gather_wait(t, b):
            for p in range(NG):
                pltpu.make_async_copy(
                    table2.at[pidx_all.at[pl.ds(t * S + p * GCHUNK, GCHUNK)]],
                    grows.at[b, pl.ds(p * GCHUNK, GCHUNK)],
                    gsems[b]).wait()

        def res_drain(t):
            pltpu.make_async_copy(
                res, out3.at[pl.ds(base_b + t, 1)], osem).wait()

        gather_start(0, 0)

        @pl.loop(0, BPW // 2)
        def _pipe(it):
            t0 = it * 2
            for b in range(2):
                t = t0 + b
                gather_start(lax.rem(t + 1, BPW), 1 - b)
                gather_wait(t, b)
                @pl.when(t >= 1)
                def _drain():
                    res_drain(t - 1)

                for r0 in GOFFS:
                    hv = (idx_all[pl.ds(t * S + r0, LANES)] & 1) * E
                    for i in range(LANES):
                        r = r0 + i
                        coff = hv[i]
                        for j in range(E // LANES):
                            sl = pl.ds(j * LANES, LANES)
                            res[0, r, sl] = (
                                grows[b, r, pl.ds(coff + j * LANES, LANES)]
                                + pos_v[r, sl])

                pltpu.async_copy(res, out3.at[pl.ds(base_b + t, 1)], osem)

        gather_wait(0, BPW % 2)
        res_drain(BPW - 1)

    return k


def kernel(x, table, pos_encoding):
    B, S = x.shape
    V, E = table.shape
    x1 = x.astype(jnp.int32).reshape(-1)
    table2 = _repack_tc(table)
    pos_s = pos_encoding[0, :S, :]
    return _build_sc_kernel(B, S, E, V)(x1, table2, pos_s)


# speedup vs baseline: 1.1975x; 1.1975x over previous
import functools

import jax
import jax.numpy as jnp
from jax import lax
from jax.experimental import pallas as pl
from jax.experimental.pallas import tpu as pltpu
from jax.experimental.pallas import tpu_sc as plsc

NUM_CORES = 2
NUM_SUBCORES = 16
NUM_WORKERS = NUM_CORES * NUM_SUBCORES
LANES = 16
IDX_CHUNK = 100


def _build_sc_kernel(B, S, E, V):
    BPW = B // NUM_WORKERS
    PAIRS = S // IDX_CHUNK

    mesh = plsc.VectorSubcoreMesh(
        core_axis_name="c", subcore_axis_name="s",
        num_cores=NUM_CORES, num_subcores=NUM_SUBCORES)

    @functools.partial(
        pl.kernel,
        out_type=jax.ShapeDtypeStruct((B, S, E), jnp.float32),
        mesh=mesh,
        scratch_types=[
            pltpu.VMEM((BPW * PAIRS, IDX_CHUNK), jnp.int32),
            pltpu.VMEM((2, S, E), jnp.float32),
            pltpu.VMEM((S, E), jnp.float32),
            pltpu.SemaphoreType.DMA,
            pltpu.SemaphoreType.DMA,
            pltpu.SemaphoreType.DMA,
        ],
        compiler_params=pltpu.CompilerParams(use_tc_tiling_on_sc=False),
    )
    def k(x2, table, pos_hbm, out3, idx_all, grows, pos_v,
          gsem0, gsem1, osem):
        c = lax.axis_index("c")
        s = lax.axis_index("s")
        wid = s * NUM_CORES + c
        base_b = pl.multiple_of(wid * BPW, BPW)
        gsems = (gsem0, gsem1)
        pltpu.sync_copy(pos_hbm, pos_v)
        pltpu.sync_copy(x2.at[pl.ds(base_b * PAIRS, BPW * PAIRS)], idx_all)

        def gather_start(t, b):
            for p in range(PAIRS):
                pltpu.async_copy(
                    table.at[idx_all.at[t * PAIRS + p]],
                    grows.at[b, pl.ds(p * IDX_CHUNK, IDX_CHUNK)],
                    gsems[b])

        def gather_wait(t, b):
            for p in range(PAIRS):
                pltpu.make_async_copy(
                    table.at[idx_all.at[t * PAIRS + p]],
                    grows.at[b, pl.ds(p * IDX_CHUNK, IDX_CHUNK)],
                    gsems[b]).wait()

        def out_write(t, b):
            return pltpu.async_copy(grows.at[b], out3.at[base_b + t], osem)

        def out_drain(t, b):
            pltpu.make_async_copy(grows.at[b], out3.at[base_b + t],
                                  osem).wait()

        gather_start(0, 0)

        @pl.loop(0, BPW // 2)
        def _pipe(it):
            t0 = it * 2
            for b in range(2):
                t = t0 + b
                @pl.when(t >= 1)
                def _drain():
                    out_drain(t - 1, 1 - b)

                gather_start(lax.rem(t + 1, BPW), 1 - b)
                gather_wait(t, b)

                @pl.loop(0, S)
                def _row(r):
                    for j in range(E // LANES):
                        sl = pl.ds(j * LANES, LANES)
                        grows[b, r, sl] = grows[b, r, sl] + pos_v[r, sl]

                out_write(t, b)

        gather_wait(0, BPW % 2)
        out_drain(BPW - 1, (BPW - 1) % 2)

    return k


def kernel(x, table, pos_encoding):
    B, S = x.shape
    V, E = table.shape
    x2 = x.astype(jnp.int32).reshape(-1, IDX_CHUNK)
    pos_s = pos_encoding[0, :S, :]
    return _build_sc_kernel(B, S, E, V)(x2, table, pos_s)


# speedup vs baseline: 1.2124x; 1.0124x over previous
import functools

import jax
import jax.numpy as jnp
from jax import lax
from jax.experimental import pallas as pl
from jax.experimental.pallas import tpu as pltpu
from jax.experimental.pallas import tpu_sc as plsc

NUM_CORES = 2
NUM_SUBCORES = 16
NUM_WORKERS = NUM_CORES * NUM_SUBCORES
LANES = 16
SUB = 40
BLK = 8


def _group_offsets(n):
    offs = list(range(0, n - LANES + 1, LANES))
    if n % LANES:
        offs.append(n - LANES)
    return offs


def _build_sc_kernel(B, S, E, V):
    BPW = B // NUM_WORKERS
    RPW = BPW * S
    NSUB = S // SUB
    TSUB = BPW * NSUB

    GROUPS = []
    for goff in _group_offsets(SUB):
        lo = 0 if goff % LANES == 0 else (SUB // LANES) * LANES - goff
        GROUPS.append((goff, lo))

    mesh = plsc.VectorSubcoreMesh(
        core_axis_name="c", subcore_axis_name="s",
        num_cores=NUM_CORES, num_subcores=NUM_SUBCORES)

    @functools.partial(
        pl.kernel,
        out_type=jax.ShapeDtypeStruct((B, S, E), jnp.float32),
        mesh=mesh,
        scratch_types=[
            pltpu.VMEM((RPW,), jnp.int32),
            pltpu.VMEM((2, SUB * BLK, E), jnp.float32),
            pltpu.VMEM((1, S, E), jnp.float32),
            pltpu.VMEM((S * E,), jnp.float32),
            pltpu.SemaphoreType.DMA,
            pltpu.SemaphoreType.DMA,
            pltpu.SemaphoreType.DMA,
        ],
    )
    def k(x1, table, pos_hbm, out3, idx_all, gblk, res, pos_v,
          gsem0, gsem1, osem):
        c = lax.axis_index("c")
        s = lax.axis_index("s")
        wid = s * NUM_CORES + c
        base = pl.multiple_of(wid * RPW, RPW)
        base_b = pl.multiple_of(wid * BPW, BPW)
        gsems = (gsem0, gsem1)
        pltpu.sync_copy(pos_hbm, pos_v)
        pltpu.sync_copy(x1.at[pl.ds(base, RPW)], idx_all)

        def fetch_start(u, b):
            u0 = u * SUB
            for goff, lo in GROUPS:
                blkv = jax.lax.shift_right_logical(
                    idx_all[pl.ds(u0 + goff, LANES)], 3)
                for i in range(lo, LANES):
                    r0 = pl.multiple_of(blkv[i] * BLK, BLK)
                    pltpu.async_copy(
                        table.at[pl.ds(r0, BLK)],
                        gblk.at[b, pl.ds((goff + i) * BLK, BLK)],
                        gsems[b])

        def fetch_drain(b):
            pltpu.make_async_copy(
                table.at[pl.ds(0, SUB * BLK)], gblk.at[b], gsems[b]).wait()

        def res_drain(t):
            pltpu.make_async_copy(
                res, out3.at[pl.ds(base_b + t, 1)], osem).wait()

        fetch_start(0, 0)

        @pl.loop(0, TSUB // 2)
        def _pipe(it):
            u0 = it * 2
            for b in range(2):
                u = u0 + b
                fetch_start(lax.rem(u + 1, TSUB), 1 - b)
                fetch_drain(b)
                @pl.when((lax.rem(u, NSUB) == 0) & (u >= 1))
                def _drain():
                    res_drain(u // NSUB - 1)

                soff = lax.rem(u, NSUB) * SUB
                for goff, lo in GROUPS:
                    hv = idx_all[pl.ds(u * SUB + goff, LANES)] & (BLK - 1)
                    for i in range(lo, LANES):
                        r = soff + goff + i
                        h = hv[i]
                        for j in range(E // LANES):
                            res[0, r, pl.ds(j * LANES, LANES)] = (
                                gblk[b, (goff + i) * BLK + h,
                                     pl.ds(j * LANES, LANES)]
                                + pos_v[pl.ds(r * E + j * LANES, LANES)])

                @pl.when(lax.rem(u, NSUB) == NSUB - 1)
                def _write():
                    pltpu.async_copy(
                        res, out3.at[pl.ds(base_b + u // NSUB, 1)], osem)

        fetch_drain(TSUB % 2)
        res_drain(BPW - 1)

    return k


def kernel(x, table, pos_encoding):
    B, S = x.shape
    V, E = table.shape
    x1 = x.astype(jnp.int32).reshape(-1)
    pos_s = pos_encoding[0, :S, :].reshape(-1)
    return _build_sc_kernel(B, S, E, V)(x1, table, pos_s)
